# baseline (speedup 1.0000x reference)
import jax
import jax.numpy as jnp
from jax import lax
from jax.experimental import pallas as pl
from jax.experimental.pallas import tpu as pltpu

_BLK = 2000
_G = 256
_GB = 64
_NCH = (_G + _GB - 1) // _GB


def _gap_kernel(lo8_ref, hi_ref, h_ref, seg_ref, W1_ref, b1_ref, W2T_ref,
                out_ref, s_ref, ex_ref):
    i = pl.program_id(0)
    j = pl.program_id(1)
    nblk = pl.num_programs(0)

    @pl.when((i == 0) & (j == 0))
    def _init():
        s_ref[...] = jnp.zeros_like(s_ref)
        out_ref[...] = jnp.zeros_like(out_ref)

    @pl.when(j == 0)
    def _gate():
        h = h_ref[...]
        u = jnp.tanh(
            lax.dot_general(h, W1_ref[...], (((1,), (0,)), ((), ())),
                            preferred_element_type=jnp.float32) + b1_ref[...])
        logits = lax.dot_general(W2T_ref[...], u, (((1,), (1,)), ((), ())),
                                 preferred_element_type=jnp.float32)
        bound = jnp.sum(jnp.abs(W2T_ref[...]), axis=1, keepdims=True)
        ex_ref[...] = jnp.exp(logits - bound)

    start = lo8_ref[i] + _GB * j

    @pl.when(start <= hi_ref[i])
    def _band():
        c = pl.multiple_of(jnp.minimum(start, _G - _GB), 8)
        gid = c + lax.broadcasted_iota(jnp.int32, (_GB, 1), 0)
        seg = seg_ref[0]
        m = (seg == gid) & (gid >= start)
        w = jnp.where(m, ex_ref[...], 0.0)
        out_ref[pl.ds(c, _GB), :] += lax.dot_general(
            w, h_ref[...], (((1,), (0,)), ((), ())),
            preferred_element_type=jnp.float32)
        s_ref[pl.ds(c, _GB), :] += jnp.sum(w, axis=1, keepdims=True)

    @pl.when((i == nblk - 1) & (j == _NCH - 1))
    def _fin():
        s = s_ref[...]
        out_ref[...] = jnp.where(s > 0.0, out_ref[...] / s, 0.0)


def _pallas_gap(lo8, hi, h, seg, W1, b1r, W2T, *, interpret=False):
    n, d = h.shape
    hdim = W1.shape[1]
    nblk = n // _BLK
    return pl.pallas_call(
        _gap_kernel,
        grid=(nblk, _NCH),
        in_specs=[
            pl.BlockSpec(memory_space=pltpu.SMEM),
            pl.BlockSpec(memory_space=pltpu.SMEM),
            pl.BlockSpec((_BLK, d), lambda i, j: (i, 0)),
            pl.BlockSpec((1, 1, _BLK), lambda i, j: (i, 0, 0)),
            pl.BlockSpec((d, hdim), lambda i, j: (0, 0)),
            pl.BlockSpec((1, hdim), lambda i, j: (0, 0)),
            pl.BlockSpec((1, hdim), lambda i, j: (0, 0)),
        ],
        out_specs=pl.BlockSpec((_G, d), lambda i, j: (0, 0)),
        out_shape=jax.ShapeDtypeStruct((_G, d), jnp.float32),
        scratch_shapes=[
            pltpu.VMEM((_G, 1), jnp.float32),
            pltpu.VMEM((1, _BLK), jnp.float32),
        ],
        interpret=interpret,
    )(lo8, hi, h, seg, W1, b1r, W2T)


@jax.jit
def kernel(h, batch, W1, b1, W2, b2):
    n = h.shape[0]
    nblk = n // _BLK
    seg1 = batch.astype(jnp.int32)
    seg = seg1.reshape(nblk, 1, _BLK)
    lo8 = (seg1[::_BLK] // 8) * 8
    hi = seg1[_BLK - 1::_BLK]
    del b2
    return _pallas_gap(lo8, hi, h, seg, W1, b1.reshape(1, -1),
                       W2.reshape(1, -1))


# speedup vs baseline: 1.9478x; 1.9478x over previous
import jax
import jax.numpy as jnp
from jax import lax
from jax.experimental import pallas as pl
from jax.experimental.pallas import tpu as pltpu

_BLK = 2000
_G = 256
_GB = 64


def _gap_kernel(lo8_ref, hi_ref, h_ref, seg_ref, W1_ref, b1_ref, W2T_ref,
                out_ref, s_ref):
    i = pl.program_id(0)
    nblk = pl.num_programs(0)

    @pl.when(i == 0)
    def _init():
        s_ref[...] = jnp.zeros_like(s_ref)
        out_ref[...] = jnp.zeros_like(out_ref)

    h = h_ref[...]
    u = jnp.tanh(
        lax.dot_general(h, W1_ref[...], (((1,), (0,)), ((), ())),
                        preferred_element_type=jnp.float32) + b1_ref[...])
    logits = lax.dot_general(W2T_ref[...], u, (((1,), (1,)), ((), ())),
                             preferred_element_type=jnp.float32)
    bound = jnp.sum(jnp.abs(W2T_ref[...]), axis=1, keepdims=True)
    ex = jnp.exp(logits - bound)
    seg = seg_ref[0]

    lo8 = lo8_ref[i]
    hi = hi_ref[i]
    nch = (hi - lo8) // _GB + 1

    def _band(j, carry):
        start = lo8 + _GB * j
        c = pl.multiple_of(jnp.minimum(start, _G - _GB), 8)
        gid = c + lax.broadcasted_iota(jnp.int32, (_GB, 1), 0)
        m = (seg == gid) & (gid >= start)
        w = jnp.where(m, ex, 0.0)
        out_ref[pl.ds(c, _GB), :] += lax.dot_general(
            w, h, (((1,), (0,)), ((), ())),
            preferred_element_type=jnp.float32)
        s_ref[pl.ds(c, _GB), :] += jnp.sum(w, axis=1, keepdims=True)
        return carry

    lax.fori_loop(0, nch, _band, 0)

    @pl.when(i == nblk - 1)
    def _fin():
        s = s_ref[...]
        out_ref[...] = jnp.where(s > 0.0, out_ref[...] / s, 0.0)


def _pallas_gap(lo8, hi, h, seg, W1, b1r, W2T, *, interpret=False):
    n, d = h.shape
    hdim = W1.shape[1]
    nblk = n // _BLK
    return pl.pallas_call(
        _gap_kernel,
        grid=(nblk,),
        in_specs=[
            pl.BlockSpec(memory_space=pltpu.SMEM),
            pl.BlockSpec(memory_space=pltpu.SMEM),
            pl.BlockSpec((_BLK, d), lambda i: (i, 0)),
            pl.BlockSpec((1, 1, _BLK), lambda i: (i, 0, 0)),
            pl.BlockSpec((d, hdim), lambda i: (0, 0)),
            pl.BlockSpec((1, hdim), lambda i: (0, 0)),
            pl.BlockSpec((1, hdim), lambda i: (0, 0)),
        ],
        out_specs=pl.BlockSpec((_G, d), lambda i: (0, 0)),
        out_shape=jax.ShapeDtypeStruct((_G, d), jnp.float32),
        scratch_shapes=[
            pltpu.VMEM((_G, 1), jnp.float32),
        ],
        interpret=interpret,
    )(lo8, hi, h, seg, W1, b1r, W2T)


@jax.jit
def kernel(h, batch, W1, b1, W2, b2):
    n = h.shape[0]
    nblk = n // _BLK
    seg1 = batch.astype(jnp.int32)
    seg = seg1.reshape(nblk, 1, _BLK)
    lo8 = (seg1[::_BLK] // 8) * 8
    hi = seg1[_BLK - 1::_BLK]
    del b2
    return _pallas_gap(lo8, hi, h, seg, W1, b1.reshape(1, -1),
                       W2.reshape(1, -1))


# speedup vs baseline: 1.9577x; 1.0051x over previous
import jax
import jax.numpy as jnp
from jax import lax
from jax.experimental import pallas as pl
from jax.experimental.pallas import tpu as pltpu

_BLK = 2000
_G = 256


def _gap_kernel(h_ref, seg_ref, W1_ref, b1_ref, W2T_ref, out_ref, s_ref):
    i = pl.program_id(0)
    nblk = pl.num_programs(0)

    @pl.when(i == 0)
    def _init():
        s_ref[...] = jnp.zeros_like(s_ref)
        out_ref[...] = jnp.zeros_like(out_ref)

    h = h_ref[...]
    hb = h.astype(jnp.bfloat16)
    seg = seg_ref[0]

    u = jnp.tanh(
        lax.dot_general(hb, W1_ref[...], (((1,), (0,)), ((), ())),
                        preferred_element_type=jnp.float32) + b1_ref[...])
    logits = lax.dot_general(W2T_ref[...], u.astype(jnp.bfloat16),
                             (((1,), (1,)), ((), ())),
                             preferred_element_type=jnp.float32)
    bound = jnp.sum(jnp.abs(W2T_ref[...].astype(jnp.float32)),
                    axis=1, keepdims=True)
    ex = jnp.exp(logits - bound)

    gid = lax.broadcasted_iota(jnp.int32, (_G, 1), 0).astype(jnp.bfloat16)
    w = jnp.where(seg == gid, ex.astype(jnp.bfloat16),
                  jnp.bfloat16(0.0))

    out_ref[...] += lax.dot_general(w, hb, (((1,), (0,)), ((), ())),
                                    preferred_element_type=jnp.float32)
    s_ref[...] += jnp.sum(w.astype(jnp.float32), axis=1, keepdims=True)

    @pl.when(i == nblk - 1)
    def _fin():
        s = s_ref[...]
        out_ref[...] = jnp.where(s > 0.0, out_ref[...] / s, 0.0)


def _pallas_gap(h, seg, W1, b1r, W2T, *, interpret=False):
    n, d = h.shape
    hdim = W1.shape[1]
    nblk = n // _BLK
    return pl.pallas_call(
        _gap_kernel,
        grid=(nblk,),
        in_specs=[
            pl.BlockSpec((_BLK, d), lambda i: (i, 0)),
            pl.BlockSpec((1, 1, _BLK), lambda i: (i, 0, 0)),
            pl.BlockSpec((d, hdim), lambda i: (0, 0)),
            pl.BlockSpec((1, hdim), lambda i: (0, 0)),
            pl.BlockSpec((1, hdim), lambda i: (0, 0)),
        ],
        out_specs=pl.BlockSpec((_G, d), lambda i: (0, 0)),
        out_shape=jax.ShapeDtypeStruct((_G, d), jnp.float32),
        scratch_shapes=[
            pltpu.VMEM((_G, 1), jnp.float32),
        ],
        interpret=interpret,
    )(h, seg, W1, b1r, W2T)


@jax.jit
def kernel(h, batch, W1, b1, W2, b2):
    n = h.shape[0]
    nblk = n // _BLK
    seg = batch.astype(jnp.int32).astype(jnp.bfloat16).reshape(nblk, 1, _BLK)
    del b2
    return _pallas_gap(h, seg, W1.astype(jnp.bfloat16), b1.reshape(1, -1),
                       W2.reshape(1, -1).astype(jnp.bfloat16))
